# Initial kernel scaffold; baseline (speedup 1.0000x reference)
import jax
import jax.numpy as jnp
from jax.experimental import pallas as pl


def kernel(x, edge_index, batch, W1, b1, W2, b2):
    raise NotImplementedError("write your pallas kernel here")


# speedup vs baseline: 19.2098x; 19.2098x over previous
import functools

import jax
import jax.numpy as jnp
from jax import lax
from jax.experimental import pallas as pl
from jax.experimental.pallas import tpu as pltpu
from jax.experimental.pallas import tpu_sc as plsc

N_NODES = 50000
N_EDGES = 800000
N_GRAPHS = 128
HID = 64

NPAD = 51200
EPAD = 819200
CHUNK = 128
NCH = EPAD // CHUNK
G = 4
NC = 2
NS = 16
RS = NPAD // NS
BLK = 1024
NBLK = NPAD // BLK

_HI = jax.lax.Precision.HIGHEST

_sc_mesh = plsc.VectorSubcoreMesh(core_axis_name="c", subcore_axis_name="s")


IDB = 1600
EPT = EPAD // (NC * NS)


@functools.partial(
    pl.kernel,
    out_type=jax.ShapeDtypeStruct((NC, NPAD), jnp.float32),
    mesh=_sc_mesh,
    scratch_types=[
        pltpu.VMEM((IDB,), jnp.int32),
        pltpu.VMEM((NPAD,), jnp.float32),
        pltpu.VMEM_SHARED((NS, NPAD), jnp.float32),
        pltpu.VMEM((RS,), jnp.float32),
        pltpu.VMEM((RS,), jnp.float32),
    ],
    compiler_params=pltpu.CompilerParams(needs_layout_passes=False),
)
def _deg(dst_ref, out_ref, ids, degloc, part, tmp, acc):
    c = lax.axis_index("c")
    s = lax.axis_index("s")
    w = c * NS + s
    z16 = jnp.zeros((16,), jnp.float32)

    def zero_body(i, carry):
        degloc[pl.ds(i * 16, 16)] = z16
        return carry

    lax.fori_loop(0, NPAD // 16, zero_body, 0)

    ones = jnp.ones((16,), jnp.float32)
    base = w * EPT

    def blk_body(i, carry):
        pltpu.sync_copy(dst_ref.at[pl.ds(base + i * IDB, IDB)], ids)

        def inner(k, carry2):
            idx = ids[pl.ds(k * 16, 16)]
            plsc.addupdate_scatter(degloc, [idx], ones)
            return carry2

        lax.fori_loop(0, IDB // 16, inner, 0)
        return carry

    lax.fori_loop(0, EPT // IDB, blk_body, 0)

    pltpu.sync_copy(degloc, part.at[s])
    plsc.subcore_barrier()
    rb = s * RS
    pltpu.sync_copy(part.at[0, pl.ds(rb, RS)], acc)

    def red_body(t, carry):
        pltpu.sync_copy(part.at[t, pl.ds(rb, RS)], tmp)

        def vadd(i, carry2):
            acc[pl.ds(i * 16, 16)] = acc[pl.ds(i * 16, 16)] + tmp[pl.ds(i * 16, 16)]
            return carry2

        lax.fori_loop(0, RS // 16, vadd, 0)
        return carry

    lax.fori_loop(1, NS, red_body, 0)
    pltpu.sync_copy(acc, out_ref.at[c, pl.ds(rb, RS)])


def _edge_loop(src_ref, dst_ref, gather_ref, acc_sh, sidx, didx, rows, gsem,
               c0, ntile):
    def grp_body(j, carry):
        cb = c0 + j * G
        pltpu.sync_copy(src_ref.at[pl.ds(cb, G)], sidx)
        pltpu.sync_copy(dst_ref.at[pl.ds(cb, G)], didx)
        cps = [
            pltpu.async_copy(gather_ref.at[sidx.at[b]], rows.at[b], gsem)
            for b in range(G)
        ]
        for cp in cps:
            cp.wait()
        for b in range(G):
            pltpu.sync_copy(rows.at[b], acc_sh.at[didx.at[b]], add=True)
        return carry

    lax.fori_loop(0, ntile // G, grp_body, 0)


@functools.partial(
    pl.kernel,
    out_type=jax.ShapeDtypeStruct((NC, NPAD, 16), jnp.float32),
    mesh=_sc_mesh,
    scratch_types=[
        pltpu.VMEM((G, CHUNK), jnp.int32),
        pltpu.VMEM((G, CHUNK), jnp.int32),
        pltpu.VMEM((G, CHUNK, 16), jnp.float32),
        pltpu.VMEM_SHARED((NPAD, 16), jnp.float32),
        pltpu.SemaphoreType.DMA,
    ],
    compiler_params=pltpu.CompilerParams(use_tc_tiling_on_sc=False),
)
def _prop16(src_ref, dst_ref, q_ref, z_ref, out_ref, sidx, didx, rows, acc_sh,
            gsem):
    c = lax.axis_index("c")
    s = lax.axis_index("s")
    rb = s * RS

    @pl.when(c == 0)
    def _():
        pltpu.sync_copy(q_ref.at[pl.ds(rb, RS)], acc_sh.at[pl.ds(rb, RS)])

    @pl.when(c == 1)
    def _():
        pltpu.sync_copy(z_ref.at[pl.ds(rb, RS)], acc_sh.at[pl.ds(rb, RS)])

    plsc.subcore_barrier()
    per_core = NCH // NC
    per_tile = per_core // NS
    c0 = c * per_core + s * per_tile
    _edge_loop(src_ref, dst_ref, q_ref, acc_sh, sidx, didx, rows, gsem,
               c0, per_tile)
    plsc.subcore_barrier()
    pltpu.sync_copy(acc_sh.at[pl.ds(rb, RS)], out_ref.at[c, pl.ds(rb, RS)])


@functools.partial(
    pl.kernel,
    out_type=jax.ShapeDtypeStruct((NC, NPAD, 32), jnp.float32),
    mesh=_sc_mesh,
    scratch_types=[
        pltpu.VMEM((G, CHUNK), jnp.int32),
        pltpu.VMEM((G, CHUNK), jnp.int32),
        pltpu.VMEM((G, CHUNK, 32), jnp.float32),
        pltpu.VMEM_SHARED((NPAD, 32), jnp.float32),
        pltpu.SemaphoreType.DMA,
    ],
    compiler_params=pltpu.CompilerParams(use_tc_tiling_on_sc=False),
)
def _prop32(src_ref, dst_ref, q_ref, out_ref, sidx, didx, rows, acc_sh, gsem):
    c = lax.axis_index("c")
    s = lax.axis_index("s")
    rb = s * RS
    pltpu.sync_copy(q_ref.at[c, pl.ds(rb, RS)], acc_sh.at[pl.ds(rb, RS)])
    plsc.subcore_barrier()
    per_tile = NCH // NS
    c0 = s * per_tile
    _edge_loop(src_ref, dst_ref, q_ref.at[c], acc_sh, sidx, didx, rows, gsem,
               c0, per_tile)
    plsc.subcore_barrier()
    pltpu.sync_copy(acc_sh.at[pl.ds(rb, RS)], out_ref.at[c, pl.ds(rb, RS)])


def _prep_body(degp_ref, x_ref, dinv_ref, q1_ref):
    deg = degp_ref[0] + degp_ref[1] + 1.0
    dinv = lax.rsqrt(deg)
    dinv_ref[...] = dinv
    q1_ref[...] = x_ref[...] * dinv


_prep = pl.pallas_call(
    _prep_body,
    grid=(NBLK,),
    in_specs=[
        pl.BlockSpec((NC, BLK, 1), lambda i: (0, i, 0)),
        pl.BlockSpec((BLK, 16), lambda i: (i, 0)),
    ],
    out_specs=[
        pl.BlockSpec((BLK, 1), lambda i: (i, 0)),
        pl.BlockSpec((BLK, 16), lambda i: (i, 0)),
    ],
    out_shape=[
        jax.ShapeDtypeStruct((NPAD, 1), jnp.float32),
        jax.ShapeDtypeStruct((NPAD, 16), jnp.float32),
    ],
)


def _mm1_body(o1_ref, dinv_ref, w_ref, b_ref, out_ref):
    dinv = dinv_ref[...]
    p = (o1_ref[0] + o1_ref[1]) * dinv
    h = jnp.dot(p, w_ref[...], preferred_element_type=jnp.float32,
                precision=_HI) + b_ref[...]
    g = jnp.maximum(h, 0.0) * dinv
    out_ref[0] = g[:, :32]
    out_ref[1] = g[:, 32:]


_mm1 = pl.pallas_call(
    _mm1_body,
    grid=(NBLK,),
    in_specs=[
        pl.BlockSpec((NC, BLK, 16), lambda i: (0, i, 0)),
        pl.BlockSpec((BLK, 1), lambda i: (i, 0)),
        pl.BlockSpec((16, HID), lambda i: (0, 0)),
        pl.BlockSpec((1, HID), lambda i: (0, 0)),
    ],
    out_specs=pl.BlockSpec((NC, BLK, 32), lambda i: (0, i, 0)),
    out_shape=jax.ShapeDtypeStruct((NC, NPAD, 32), jnp.float32),
)


def _mm2_body(o2_ref, dinv_ref, w_ref, b_ref, bat_ref, out_ref, acc, cnt):
    i = pl.program_id(0)

    @pl.when(i == 0)
    def _():
        acc[...] = jnp.zeros_like(acc)
        cnt[...] = jnp.zeros_like(cnt)

    p = jnp.concatenate([o2_ref[0], o2_ref[1]], axis=1) * dinv_ref[...]
    h = jnp.dot(p, w_ref[...], preferred_element_type=jnp.float32,
                precision=_HI) + b_ref[...]
    h = jnp.maximum(h, 0.0)
    gids = lax.broadcasted_iota(jnp.int32, (1, N_GRAPHS), 1)
    oh = (bat_ref[...] == gids).astype(jnp.float32)
    dn = (((0,), (0,)), ((), ()))
    acc[...] += lax.dot_general(oh, h, dn, preferred_element_type=jnp.float32,
                                precision=_HI)
    cnt[...] += lax.dot_general(oh, jnp.ones((BLK, 1), jnp.float32), dn,
                                preferred_element_type=jnp.float32,
                                precision=_HI)

    @pl.when(i == NBLK - 1)
    def _():
        out_ref[...] = acc[...] / jnp.maximum(cnt[...], 1.0)


_mm2 = pl.pallas_call(
    _mm2_body,
    grid=(NBLK,),
    in_specs=[
        pl.BlockSpec((NC, BLK, 32), lambda i: (0, i, 0)),
        pl.BlockSpec((BLK, 1), lambda i: (i, 0)),
        pl.BlockSpec((HID, HID), lambda i: (0, 0)),
        pl.BlockSpec((1, HID), lambda i: (0, 0)),
        pl.BlockSpec((BLK, 1), lambda i: (i, 0)),
    ],
    out_specs=pl.BlockSpec((N_GRAPHS, HID), lambda i: (0, 0)),
    out_shape=jax.ShapeDtypeStruct((N_GRAPHS, HID), jnp.float32),
    scratch_shapes=[
        pltpu.VMEM((N_GRAPHS, HID), jnp.float32),
        pltpu.VMEM((N_GRAPHS, 1), jnp.float32),
    ],
)


def kernel(x, edge_index, batch, W1, b1, W2, b2):
    src = edge_index[0].astype(jnp.int32)
    dst = edge_index[1].astype(jnp.int32)
    pad_e = jnp.full((EPAD - N_EDGES,), N_NODES, jnp.int32)
    srcf = jnp.concatenate([src, pad_e])
    dstf = jnp.concatenate([dst, pad_e])
    src2 = srcf.reshape(NCH, CHUNK)
    dst2 = dstf.reshape(NCH, CHUNK)

    degp = _deg(dstf).reshape(NC, NPAD, 1)

    xp = jnp.zeros((NPAD, 16), jnp.float32).at[:N_NODES, :3].set(x)
    dinv2, q1 = _prep(degp, xp)

    z16 = jnp.zeros((NPAD, 16), jnp.float32)
    o1 = _prop16(src2, dst2, q1, z16)

    W1p = jnp.zeros((16, HID), jnp.float32).at[:3].set(W1)
    q2 = _mm1(o1, dinv2, W1p, b1.reshape(1, HID))

    o2 = _prop32(src2, dst2, q2)

    batp = jnp.concatenate(
        [batch.astype(jnp.int32),
         jnp.full((NPAD - N_NODES,), jnp.int32(2**30), jnp.int32)]
    ).reshape(NPAD, 1)
    out = _mm2(o2, dinv2, W2, b2.reshape(1, HID), batp)
    return out


# speedup vs baseline: 23.6246x; 1.2298x over previous
import functools

import jax
import jax.numpy as jnp
from jax import lax
from jax.experimental import pallas as pl
from jax.experimental.pallas import tpu as pltpu
from jax.experimental.pallas import tpu_sc as plsc

N_NODES = 50000
N_EDGES = 800000
N_GRAPHS = 128
HID = 64

NPAD = 51200
EPAD = 819200
CHUNK = 128
NCH = EPAD // CHUNK
NB = 4
BID = 20
NC = 2
NS = 16
RS = NPAD // NS
BLK = 1024
NBLK = NPAD // BLK

_HI = jax.lax.Precision.HIGHEST

_sc_mesh = plsc.VectorSubcoreMesh(core_axis_name="c", subcore_axis_name="s")


IDB = 1600
EPT = EPAD // (NC * NS)


@functools.partial(
    pl.kernel,
    out_type=jax.ShapeDtypeStruct((NC, NPAD), jnp.float32),
    mesh=_sc_mesh,
    scratch_types=[
        pltpu.VMEM((IDB,), jnp.int32),
        pltpu.VMEM((NPAD,), jnp.float32),
        pltpu.VMEM_SHARED((NS, NPAD), jnp.float32),
        pltpu.VMEM((RS,), jnp.float32),
        pltpu.VMEM((RS,), jnp.float32),
    ],
    compiler_params=pltpu.CompilerParams(needs_layout_passes=False),
)
def _deg(dst_ref, out_ref, ids, degloc, part, tmp, acc):
    c = lax.axis_index("c")
    s = lax.axis_index("s")
    w = c * NS + s
    z16 = jnp.zeros((16,), jnp.float32)

    def zero_body(i, carry):
        degloc[pl.ds(i * 16, 16)] = z16
        return carry

    lax.fori_loop(0, NPAD // 16, zero_body, 0)

    ones = jnp.ones((16,), jnp.float32)
    base = w * EPT

    def blk_body(i, carry):
        pltpu.sync_copy(dst_ref.at[pl.ds(base + i * IDB, IDB)], ids)

        def inner(k, carry2):
            idx = ids[pl.ds(k * 16, 16)]
            plsc.addupdate_scatter(degloc, [idx], ones)
            return carry2

        lax.fori_loop(0, IDB // 16, inner, 0)
        return carry

    lax.fori_loop(0, EPT // IDB, blk_body, 0)

    pltpu.sync_copy(degloc, part.at[s])
    plsc.subcore_barrier()
    rb = s * RS
    pltpu.sync_copy(part.at[0, pl.ds(rb, RS)], acc)

    def red_body(t, carry):
        pltpu.sync_copy(part.at[t, pl.ds(rb, RS)], tmp)

        def vadd(i, carry2):
            acc[pl.ds(i * 16, 16)] = acc[pl.ds(i * 16, 16)] + tmp[pl.ds(i * 16, 16)]
            return carry2

        lax.fori_loop(0, RS // 16, vadd, 0)
        return carry

    lax.fori_loop(1, NS, red_body, 0)
    pltpu.sync_copy(acc, out_ref.at[c, pl.ds(rb, RS)])


def _edge_loop(src_ref, dst_ref, gather_ref, acc_sh, sidx, didx, rows, gsem,
               ssem, isem, c0, ntile):
    nblk = ntile // BID

    def load_ids(ib, blk):
        cb = c0 + blk * BID
        pltpu.async_copy(src_ref.at[pl.ds(cb, BID)], sidx.at[ib], isem.at[ib])
        pltpu.async_copy(dst_ref.at[pl.ds(cb, BID)], didx.at[ib], isem.at[ib])

    def wait_ids(ib):
        for r in (sidx, didx):
            pltpu.make_async_copy(src_ref.at[pl.ds(0, BID)], r.at[ib],
                                  isem.at[ib]).wait()

    def fire_gather(ib, u, b):
        pltpu.async_copy(gather_ref.at[sidx.at[ib, u]], rows.at[b],
                         gsem.at[b])

    def drain_gather(b):
        pltpu.make_async_copy(gather_ref.at[pl.ds(0, CHUNK)], rows.at[b],
                              gsem.at[b]).wait()

    def fire_add(ib, u, b):
        pltpu.async_copy(rows.at[b], acc_sh.at[didx.at[ib, u]], ssem.at[b],
                         add=True)

    def drain_add(b):
        pltpu.make_async_copy(rows.at[b], acc_sh.at[pl.ds(0, CHUNK)],
                              ssem.at[b]).wait()

    load_ids(0, 0)

    def blk_body(blk, carry):
        ib = blk % 2
        wait_ids(ib)

        @pl.when(blk + 1 < nblk)
        def _():
            load_ids(1 - ib, blk + 1)

        for u in range(NB - 1):
            fire_gather(ib, u, u)
        for u in range(BID):
            b = u % NB
            drain_gather(b)
            fire_add(ib, u, b)
            nxt = u + NB - 1
            if nxt < BID:
                pb = nxt % NB
                if u >= 1:
                    drain_add(pb)
                fire_gather(ib, nxt, pb)
        for b in range(NB):
            drain_add(b)
        return carry

    lax.fori_loop(0, nblk, blk_body, 0)


@functools.partial(
    pl.kernel,
    out_type=jax.ShapeDtypeStruct((NC, NPAD, 16), jnp.float32),
    mesh=_sc_mesh,
    scratch_types=[
        pltpu.VMEM((2, BID, CHUNK), jnp.int32),
        pltpu.VMEM((2, BID, CHUNK), jnp.int32),
        pltpu.VMEM((NB, CHUNK, 16), jnp.float32),
        pltpu.VMEM_SHARED((NPAD, 16), jnp.float32),
        pltpu.SemaphoreType.DMA((NB,)),
        pltpu.SemaphoreType.DMA((NB,)),
        pltpu.SemaphoreType.DMA((2,)),
    ],
    compiler_params=pltpu.CompilerParams(use_tc_tiling_on_sc=False),
)
def _prop16(src_ref, dst_ref, q_ref, z_ref, out_ref, sidx, didx, rows, acc_sh,
            gsem, ssem, isem):
    c = lax.axis_index("c")
    s = lax.axis_index("s")
    rb = s * RS

    @pl.when(c == 0)
    def _():
        pltpu.sync_copy(q_ref.at[pl.ds(rb, RS)], acc_sh.at[pl.ds(rb, RS)])

    @pl.when(c == 1)
    def _():
        pltpu.sync_copy(z_ref.at[pl.ds(rb, RS)], acc_sh.at[pl.ds(rb, RS)])

    plsc.subcore_barrier()
    per_core = NCH // NC
    per_tile = per_core // NS
    c0 = c * per_core + s * per_tile
    _edge_loop(src_ref, dst_ref, q_ref, acc_sh, sidx, didx, rows, gsem,
               ssem, isem, c0, per_tile)
    plsc.subcore_barrier()
    pltpu.sync_copy(acc_sh.at[pl.ds(rb, RS)], out_ref.at[c, pl.ds(rb, RS)])


@functools.partial(
    pl.kernel,
    out_type=jax.ShapeDtypeStruct((NC, NPAD, 32), jnp.float32),
    mesh=_sc_mesh,
    scratch_types=[
        pltpu.VMEM((2, BID, CHUNK), jnp.int32),
        pltpu.VMEM((2, BID, CHUNK), jnp.int32),
        pltpu.VMEM((NB, CHUNK, 32), jnp.float32),
        pltpu.VMEM_SHARED((NPAD, 32), jnp.float32),
        pltpu.SemaphoreType.DMA((NB,)),
        pltpu.SemaphoreType.DMA((NB,)),
        pltpu.SemaphoreType.DMA((2,)),
    ],
    compiler_params=pltpu.CompilerParams(use_tc_tiling_on_sc=False),
)
def _prop32(src_ref, dst_ref, q_ref, out_ref, sidx, didx, rows, acc_sh, gsem,
            ssem, isem):
    c = lax.axis_index("c")
    s = lax.axis_index("s")
    rb = s * RS
    pltpu.sync_copy(q_ref.at[c, pl.ds(rb, RS)], acc_sh.at[pl.ds(rb, RS)])
    plsc.subcore_barrier()
    per_tile = NCH // NS
    c0 = s * per_tile
    _edge_loop(src_ref, dst_ref, q_ref.at[c], acc_sh, sidx, didx, rows, gsem,
               ssem, isem, c0, per_tile)
    plsc.subcore_barrier()
    pltpu.sync_copy(acc_sh.at[pl.ds(rb, RS)], out_ref.at[c, pl.ds(rb, RS)])


def _prep_body(degp_ref, x_ref, dinv_ref, q1_ref):
    deg = degp_ref[0] + degp_ref[1] + 1.0
    dinv = lax.rsqrt(deg)
    dinv_ref[...] = dinv
    q1_ref[...] = x_ref[...] * dinv


_prep = pl.pallas_call(
    _prep_body,
    grid=(NBLK,),
    in_specs=[
        pl.BlockSpec((NC, BLK, 1), lambda i: (0, i, 0)),
        pl.BlockSpec((BLK, 16), lambda i: (i, 0)),
    ],
    out_specs=[
        pl.BlockSpec((BLK, 1), lambda i: (i, 0)),
        pl.BlockSpec((BLK, 16), lambda i: (i, 0)),
    ],
    out_shape=[
        jax.ShapeDtypeStruct((NPAD, 1), jnp.float32),
        jax.ShapeDtypeStruct((NPAD, 16), jnp.float32),
    ],
)


def _mm1_body(o1_ref, dinv_ref, w_ref, b_ref, out_ref):
    dinv = dinv_ref[...]
    p = (o1_ref[0] + o1_ref[1]) * dinv
    h = jnp.dot(p, w_ref[...], preferred_element_type=jnp.float32,
                precision=_HI) + b_ref[...]
    g = jnp.maximum(h, 0.0) * dinv
    out_ref[0] = g[:, :32]
    out_ref[1] = g[:, 32:]


_mm1 = pl.pallas_call(
    _mm1_body,
    grid=(NBLK,),
    in_specs=[
        pl.BlockSpec((NC, BLK, 16), lambda i: (0, i, 0)),
        pl.BlockSpec((BLK, 1), lambda i: (i, 0)),
        pl.BlockSpec((16, HID), lambda i: (0, 0)),
        pl.BlockSpec((1, HID), lambda i: (0, 0)),
    ],
    out_specs=pl.BlockSpec((NC, BLK, 32), lambda i: (0, i, 0)),
    out_shape=jax.ShapeDtypeStruct((NC, NPAD, 32), jnp.float32),
)


def _mm2_body(o2_ref, dinv_ref, w_ref, b_ref, bat_ref, out_ref, acc, cnt):
    i = pl.program_id(0)

    @pl.when(i == 0)
    def _():
        acc[...] = jnp.zeros_like(acc)
        cnt[...] = jnp.zeros_like(cnt)

    p = jnp.concatenate([o2_ref[0], o2_ref[1]], axis=1) * dinv_ref[...]
    h = jnp.dot(p, w_ref[...], preferred_element_type=jnp.float32,
                precision=_HI) + b_ref[...]
    h = jnp.maximum(h, 0.0)
    gids = lax.broadcasted_iota(jnp.int32, (1, N_GRAPHS), 1)
    oh = (bat_ref[...] == gids).astype(jnp.float32)
    dn = (((0,), (0,)), ((), ()))
    acc[...] += lax.dot_general(oh, h, dn, preferred_element_type=jnp.float32,
                                precision=_HI)
    cnt[...] += lax.dot_general(oh, jnp.ones((BLK, 1), jnp.float32), dn,
                                preferred_element_type=jnp.float32,
                                precision=_HI)

    @pl.when(i == NBLK - 1)
    def _():
        out_ref[...] = acc[...] / jnp.maximum(cnt[...], 1.0)


_mm2 = pl.pallas_call(
    _mm2_body,
    grid=(NBLK,),
    in_specs=[
        pl.BlockSpec((NC, BLK, 32), lambda i: (0, i, 0)),
        pl.BlockSpec((BLK, 1), lambda i: (i, 0)),
        pl.BlockSpec((HID, HID), lambda i: (0, 0)),
        pl.BlockSpec((1, HID), lambda i: (0, 0)),
        pl.BlockSpec((BLK, 1), lambda i: (i, 0)),
    ],
    out_specs=pl.BlockSpec((N_GRAPHS, HID), lambda i: (0, 0)),
    out_shape=jax.ShapeDtypeStruct((N_GRAPHS, HID), jnp.float32),
    scratch_shapes=[
        pltpu.VMEM((N_GRAPHS, HID), jnp.float32),
        pltpu.VMEM((N_GRAPHS, 1), jnp.float32),
    ],
)


def kernel(x, edge_index, batch, W1, b1, W2, b2):
    src = edge_index[0].astype(jnp.int32)
    dst = edge_index[1].astype(jnp.int32)
    pad_e = jnp.full((EPAD - N_EDGES,), N_NODES, jnp.int32)
    srcf = jnp.concatenate([src, pad_e])
    dstf = jnp.concatenate([dst, pad_e])
    src2 = srcf.reshape(NCH, CHUNK)
    dst2 = dstf.reshape(NCH, CHUNK)

    degp = _deg(dstf).reshape(NC, NPAD, 1)

    xp = jnp.zeros((NPAD, 16), jnp.float32).at[:N_NODES, :3].set(x)
    dinv2, q1 = _prep(degp, xp)

    z16 = jnp.zeros((NPAD, 16), jnp.float32)
    o1 = _prop16(src2, dst2, q1, z16)

    W1p = jnp.zeros((16, HID), jnp.float32).at[:3].set(W1)
    q2 = _mm1(o1, dinv2, W1p, b1.reshape(1, HID))

    o2 = _prop32(src2, dst2, q2)

    batp = jnp.concatenate(
        [batch.astype(jnp.int32),
         jnp.full((NPAD - N_NODES,), jnp.int32(2**30), jnp.int32)]
    ).reshape(NPAD, 1)
    out = _mm2(o2, dinv2, W2, b2.reshape(1, HID), batp)
    return out


# speedup vs baseline: 26.5776x; 1.1250x over previous
import functools

import jax
import jax.numpy as jnp
from jax import lax
from jax.experimental import pallas as pl
from jax.experimental.pallas import tpu as pltpu
from jax.experimental.pallas import tpu_sc as plsc

N_NODES = 50000
N_EDGES = 800000
N_GRAPHS = 128
HID = 64

NPAD = 51200
EPAD = 819200
CHUNK = 128
NCH = EPAD // CHUNK
NB = 4
BID = 20
NC = 2
NS = 16
RS = NPAD // NS
BLK = 2048
NBLK = NPAD // BLK

_HI = jax.lax.Precision.DEFAULT

_sc_mesh = plsc.VectorSubcoreMesh(core_axis_name="c", subcore_axis_name="s")


IDB = 1600
EPT = EPAD // (NC * NS)


@functools.partial(
    pl.kernel,
    out_type=jax.ShapeDtypeStruct((NC, NPAD), jnp.float32),
    mesh=_sc_mesh,
    scratch_types=[
        pltpu.VMEM((IDB,), jnp.int32),
        pltpu.VMEM((NPAD,), jnp.float32),
        pltpu.VMEM_SHARED((NS, NPAD), jnp.float32),
        pltpu.VMEM((RS,), jnp.float32),
        pltpu.VMEM((RS,), jnp.float32),
    ],
    compiler_params=pltpu.CompilerParams(needs_layout_passes=False),
)
def _deg(dst_ref, out_ref, ids, degloc, part, tmp, acc):
    c = lax.axis_index("c")
    s = lax.axis_index("s")
    w = c * NS + s
    z16 = jnp.zeros((16,), jnp.float32)

    def zero_body(i, carry):
        degloc[pl.ds(i * 16, 16)] = z16
        return carry

    lax.fori_loop(0, NPAD // 16, zero_body, 0)

    ones = jnp.ones((16,), jnp.float32)
    base = w * EPT

    def blk_body(i, carry):
        pltpu.sync_copy(dst_ref.at[pl.ds(base + i * IDB, IDB)], ids)

        def inner(k, carry2):
            idx = ids[pl.ds(k * 16, 16)]
            plsc.addupdate_scatter(degloc, [idx], ones)
            return carry2

        lax.fori_loop(0, IDB // 16, inner, 0)
        return carry

    lax.fori_loop(0, EPT // IDB, blk_body, 0)

    pltpu.sync_copy(degloc, part.at[s])
    plsc.subcore_barrier()
    rb = s * RS
    pltpu.sync_copy(part.at[0, pl.ds(rb, RS)], acc)

    def red_body(t, carry):
        pltpu.sync_copy(part.at[t, pl.ds(rb, RS)], tmp)

        def vadd(i, carry2):
            acc[pl.ds(i * 16, 16)] = acc[pl.ds(i * 16, 16)] + tmp[pl.ds(i * 16, 16)]
            return carry2

        lax.fori_loop(0, RS // 16, vadd, 0)
        return carry

    lax.fori_loop(1, NS, red_body, 0)
    pltpu.sync_copy(acc, out_ref.at[c, pl.ds(rb, RS)])


def _edge_loop(src_ref, dst_ref, gather_ref, acc_sh, sidx, didx, rows, gsem,
               ssem, isem, c0, ntile):
    nblk = ntile // BID

    def load_ids(ib, blk):
        cb = c0 + blk * BID
        pltpu.async_copy(src_ref.at[pl.ds(cb, BID)], sidx.at[ib], isem.at[ib])
        pltpu.async_copy(dst_ref.at[pl.ds(cb, BID)], didx.at[ib], isem.at[ib])

    def wait_ids(ib):
        for r in (sidx, didx):
            pltpu.make_async_copy(src_ref.at[pl.ds(0, BID)], r.at[ib],
                                  isem.at[ib]).wait()

    def fire_gather(ib, u, b):
        pltpu.async_copy(gather_ref.at[sidx.at[ib, u]], rows.at[b],
                         gsem.at[b])

    def drain_gather(b):
        pltpu.make_async_copy(gather_ref.at[pl.ds(0, CHUNK)], rows.at[b],
                              gsem.at[b]).wait()

    def fire_add(ib, u, b):
        pltpu.async_copy(rows.at[b], acc_sh.at[didx.at[ib, u]], ssem.at[b],
                         add=True)

    def drain_add(b):
        pltpu.make_async_copy(rows.at[b], acc_sh.at[pl.ds(0, CHUNK)],
                              ssem.at[b]).wait()

    load_ids(0, 0)

    def blk_body(blk, carry):
        ib = blk % 2
        wait_ids(ib)

        @pl.when(blk + 1 < nblk)
        def _():
            load_ids(1 - ib, blk + 1)

        for u in range(NB - 1):
            fire_gather(ib, u, u)
        for u in range(BID):
            b = u % NB
            drain_gather(b)
            fire_add(ib, u, b)
            nxt = u + NB - 1
            if nxt < BID:
                pb = nxt % NB
                if u >= 1:
                    drain_add(pb)
                fire_gather(ib, nxt, pb)
        for b in range(NB):
            drain_add(b)
        return carry

    lax.fori_loop(0, nblk, blk_body, 0)


@functools.partial(
    pl.kernel,
    out_type=jax.ShapeDtypeStruct((NC, NPAD, 16), jnp.float32),
    mesh=_sc_mesh,
    scratch_types=[
        pltpu.VMEM((2, BID, CHUNK), jnp.int32),
        pltpu.VMEM((2, BID, CHUNK), jnp.int32),
        pltpu.VMEM((NB, CHUNK, 16), jnp.float32),
        pltpu.VMEM_SHARED((NPAD, 16), jnp.float32),
        pltpu.SemaphoreType.DMA((NB,)),
        pltpu.SemaphoreType.DMA((NB,)),
        pltpu.SemaphoreType.DMA((2,)),
    ],
    compiler_params=pltpu.CompilerParams(use_tc_tiling_on_sc=False),
)
def _prop16(src_ref, dst_ref, q_ref, z_ref, out_ref, sidx, didx, rows, acc_sh,
            gsem, ssem, isem):
    c = lax.axis_index("c")
    s = lax.axis_index("s")
    rb = s * RS

    @pl.when(c == 0)
    def _():
        pltpu.sync_copy(q_ref.at[pl.ds(rb, RS)], acc_sh.at[pl.ds(rb, RS)])

    @pl.when(c == 1)
    def _():
        pltpu.sync_copy(z_ref.at[pl.ds(rb, RS)], acc_sh.at[pl.ds(rb, RS)])

    plsc.subcore_barrier()
    per_core = NCH // NC
    per_tile = per_core // NS
    c0 = c * per_core + s * per_tile
    _edge_loop(src_ref, dst_ref, q_ref, acc_sh, sidx, didx, rows, gsem,
               ssem, isem, c0, per_tile)
    plsc.subcore_barrier()
    pltpu.sync_copy(acc_sh.at[pl.ds(rb, RS)], out_ref.at[c, pl.ds(rb, RS)])


@functools.partial(
    pl.kernel,
    out_type=jax.ShapeDtypeStruct((NC, NPAD, 32), jnp.float32),
    mesh=_sc_mesh,
    scratch_types=[
        pltpu.VMEM((2, BID, CHUNK), jnp.int32),
        pltpu.VMEM((2, BID, CHUNK), jnp.int32),
        pltpu.VMEM((NB, CHUNK, 32), jnp.float32),
        pltpu.VMEM_SHARED((NPAD, 32), jnp.float32),
        pltpu.SemaphoreType.DMA((NB,)),
        pltpu.SemaphoreType.DMA((NB,)),
        pltpu.SemaphoreType.DMA((2,)),
    ],
    compiler_params=pltpu.CompilerParams(use_tc_tiling_on_sc=False),
)
def _prop32(src_ref, dst_ref, q_ref, out_ref, sidx, didx, rows, acc_sh, gsem,
            ssem, isem):
    c = lax.axis_index("c")
    s = lax.axis_index("s")
    rb = s * RS
    pltpu.sync_copy(q_ref.at[c, pl.ds(rb, RS)], acc_sh.at[pl.ds(rb, RS)])
    plsc.subcore_barrier()
    per_tile = NCH // NS
    c0 = s * per_tile
    _edge_loop(src_ref, dst_ref, q_ref.at[c], acc_sh, sidx, didx, rows, gsem,
               ssem, isem, c0, per_tile)
    plsc.subcore_barrier()
    pltpu.sync_copy(acc_sh.at[pl.ds(rb, RS)], out_ref.at[c, pl.ds(rb, RS)])


def _prep_body(degp_ref, x_ref, dinv_ref, q1_ref):
    deg = degp_ref[0] + degp_ref[1] + 1.0
    dinv = lax.rsqrt(deg)
    dinv_ref[...] = dinv
    q1_ref[...] = x_ref[...] * dinv


_prep = pl.pallas_call(
    _prep_body,
    grid=(NBLK,),
    in_specs=[
        pl.BlockSpec((NC, BLK, 1), lambda i: (0, i, 0)),
        pl.BlockSpec((BLK, 16), lambda i: (i, 0)),
    ],
    out_specs=[
        pl.BlockSpec((BLK, 1), lambda i: (i, 0)),
        pl.BlockSpec((BLK, 16), lambda i: (i, 0)),
    ],
    out_shape=[
        jax.ShapeDtypeStruct((NPAD, 1), jnp.float32),
        jax.ShapeDtypeStruct((NPAD, 16), jnp.float32),
    ],
)


def _mm1_body(o1_ref, dinv_ref, w_ref, b_ref, out_ref):
    dinv = dinv_ref[...]
    p = (o1_ref[0] + o1_ref[1]) * dinv
    h = jnp.dot(p, w_ref[...], preferred_element_type=jnp.float32,
                precision=_HI) + b_ref[...]
    g = jnp.maximum(h, 0.0) * dinv
    out_ref[0] = g[:, :32]
    out_ref[1] = g[:, 32:]


_mm1 = pl.pallas_call(
    _mm1_body,
    grid=(NBLK,),
    in_specs=[
        pl.BlockSpec((NC, BLK, 16), lambda i: (0, i, 0)),
        pl.BlockSpec((BLK, 1), lambda i: (i, 0)),
        pl.BlockSpec((16, HID), lambda i: (0, 0)),
        pl.BlockSpec((1, HID), lambda i: (0, 0)),
    ],
    out_specs=pl.BlockSpec((NC, BLK, 32), lambda i: (0, i, 0)),
    out_shape=jax.ShapeDtypeStruct((NC, NPAD, 32), jnp.float32),
)


def _mm2_body(o2_ref, dinv_ref, w_ref, b_ref, bat_ref, out_ref, acc, cnt):
    i = pl.program_id(0)

    @pl.when(i == 0)
    def _():
        acc[...] = jnp.zeros_like(acc)
        cnt[...] = jnp.zeros_like(cnt)

    p = jnp.concatenate([o2_ref[0], o2_ref[1]], axis=1) * dinv_ref[...]
    h = jnp.dot(p, w_ref[...], preferred_element_type=jnp.float32,
                precision=_HI) + b_ref[...]
    h = jnp.maximum(h, 0.0)
    gids = lax.broadcasted_iota(jnp.int32, (1, N_GRAPHS), 1)
    oh = (bat_ref[...] == gids).astype(jnp.float32)
    dn = (((0,), (0,)), ((), ()))
    acc[...] += lax.dot_general(oh, h, dn, preferred_element_type=jnp.float32,
                                precision=_HI)
    cnt[...] += lax.dot_general(oh, jnp.ones((BLK, 1), jnp.float32), dn,
                                preferred_element_type=jnp.float32,
                                precision=_HI)

    @pl.when(i == NBLK - 1)
    def _():
        out_ref[...] = acc[...] / jnp.maximum(cnt[...], 1.0)


_mm2 = pl.pallas_call(
    _mm2_body,
    grid=(NBLK,),
    in_specs=[
        pl.BlockSpec((NC, BLK, 32), lambda i: (0, i, 0)),
        pl.BlockSpec((BLK, 1), lambda i: (i, 0)),
        pl.BlockSpec((HID, HID), lambda i: (0, 0)),
        pl.BlockSpec((1, HID), lambda i: (0, 0)),
        pl.BlockSpec((BLK, 1), lambda i: (i, 0)),
    ],
    out_specs=pl.BlockSpec((N_GRAPHS, HID), lambda i: (0, 0)),
    out_shape=jax.ShapeDtypeStruct((N_GRAPHS, HID), jnp.float32),
    scratch_shapes=[
        pltpu.VMEM((N_GRAPHS, HID), jnp.float32),
        pltpu.VMEM((N_GRAPHS, 1), jnp.float32),
    ],
)


def kernel(x, edge_index, batch, W1, b1, W2, b2):
    src = edge_index[0].astype(jnp.int32)
    dst = edge_index[1].astype(jnp.int32)
    pad_e = jnp.full((EPAD - N_EDGES,), N_NODES, jnp.int32)
    srcf = jnp.concatenate([src, pad_e])
    dstf = jnp.concatenate([dst, pad_e])
    src2 = srcf.reshape(NCH, CHUNK)
    dst2 = dstf.reshape(NCH, CHUNK)

    degp = _deg(dstf).reshape(NC, NPAD, 1)

    xp = jnp.zeros((NPAD, 16), jnp.float32).at[:N_NODES, :3].set(x)
    dinv2, q1 = _prep(degp, xp)

    z16 = jnp.zeros((NPAD, 16), jnp.float32)
    o1 = _prop16(src2, dst2, q1, z16)

    W1p = jnp.zeros((16, HID), jnp.float32).at[:3].set(W1)
    q2 = _mm1(o1, dinv2, W1p, b1.reshape(1, HID))

    o2 = _prop32(src2, dst2, q2)

    batp = jnp.concatenate(
        [batch.astype(jnp.int32),
         jnp.full((NPAD - N_NODES,), jnp.int32(2**30), jnp.int32)]
    ).reshape(NPAD, 1)
    out = _mm2(o2, dinv2, W2, b2.reshape(1, HID), batp)
    return out
